# Initial kernel scaffold; baseline (speedup 1.0000x reference)
import jax
import jax.numpy as jnp
from jax.experimental import pallas as pl


def kernel(decoder_output, input_ids, segmentation_indices, W_in, b_in, ln1_g, ln1_b, W1, b1, conv_k, A_log, W2, b2, ln2_g, ln2_b, Wm1, bm1, Wm2, bm2, lnf_g, lnf_b, W_mu, b_mu, W_lv, b_lv):
    raise NotImplementedError("write your pallas kernel here")


# speedup vs baseline: 33.2573x; 33.2573x over previous
import jax
import jax.numpy as jnp
from jax import lax
from jax.experimental import pallas as pl
from jax.experimental.pallas import tpu as pltpu

B, L = 4, 2048
HIDDEN = 512
LATENT = 128
NUM_LAYERS = 2
CONV_LEN = 4
D_INNER = LATENT * 2
MLP_INNER = 256
SEG = 256
N = L // SEG
ROWS = 2 * B
TOK = ROWS * SEG


def _layernorm(x, g, b):
    m = x.mean(-1, keepdims=True)
    xc = x - m
    v = (xc * xc).mean(-1, keepdims=True)
    return xc * lax.rsqrt(v + 1e-5) * g + b


def _silu(x):
    return x * jax.nn.sigmoid(x)


def _mixer_body(ids_ref, dec_ref, gate_ref, W_in_ref, b_in_ref, ln1_g_ref,
                ln1_b_ref, W1_ref, b1_ref, conv_k_ref, A_log_ref, W2_ref,
                b2_ref, ln2_g_ref, ln2_b_ref, Wm1_ref, bm1_ref, Wm2_ref,
                bm2_ref, lnf_g_ref, lnf_b_ref, W_mu_ref, b_mu_ref, W_lv_ref,
                b_lv_ref, mu_ref, lv_ref, tail_scr, ssm_scr):
    i = pl.program_id(0)

    @pl.when(i == 0)
    def _init():
        tail_scr[...] = jnp.zeros_like(tail_scr)
        ssm_scr[...] = jnp.zeros_like(ssm_scr)

    xin = jnp.concatenate([ids_ref[...], dec_ref[...]], axis=0)
    xin = xin.reshape(TOK, HIDDEN)
    x = jnp.dot(xin, W_in_ref[...], preferred_element_type=jnp.float32)
    x = (x + b_in_ref[...]).reshape(ROWS, SEG, LATENT)

    for l in range(NUM_LAYERS):
        h = _layernorm(x, ln1_g_ref[l], ln1_b_ref[l])
        xz = jnp.dot(h.reshape(TOK, LATENT), W1_ref[l],
                     preferred_element_type=jnp.float32) + b1_ref[l]
        xz = xz.reshape(ROWS, SEG, 2 * D_INNER)
        u = xz[:, :, :D_INNER]
        z = xz[:, :, D_INNER:]

        tail = tail_scr[l]
        tail8 = jnp.concatenate([tail, tail], axis=0)
        up = jnp.concatenate([tail8, u], axis=1)
        ck = conv_k_ref[l]
        uc = (up[:, 0:SEG, :] * ck[0] + up[:, 1:SEG + 1, :] * ck[1]
              + up[:, 2:SEG + 2, :] * ck[2] + up[:, 3:SEG + 3, :] * ck[3])
        tail_scr[l] = u[:B, SEG - (CONV_LEN - 1):, :]
        ua = _silu(uc)

        la = jnp.exp(A_log_ref[l])
        S = ua
        d = 1
        while d < SEG:
            shifted = jnp.concatenate(
                [jnp.zeros((ROWS, d, D_INNER), jnp.float32),
                 S[:, :SEG - d, :]], axis=1)
            S = S + jnp.exp(-float(d) * la) * shifted
            d *= 2
        tpow = lax.broadcasted_iota(
            jnp.int32, (SEG, D_INNER), 0).astype(jnp.float32) + 1.0
        decay = jnp.exp(-tpow * la)
        sp = ssm_scr[l].reshape(B, 1, D_INNER)
        sp8 = jnp.concatenate([sp, sp], axis=0)
        y = S + decay[None, :, :] * sp8
        ssm_scr[l] = y[:B, SEG - 1, :]

        g2 = (y * _silu(z)).reshape(TOK, D_INNER)
        x = x + (jnp.dot(g2, W2_ref[l], preferred_element_type=jnp.float32)
                 + b2_ref[l]).reshape(ROWS, SEG, LATENT)
        h2 = _layernorm(x, ln2_g_ref[l], ln2_b_ref[l])
        mm = jax.nn.gelu(jnp.dot(h2.reshape(TOK, LATENT), Wm1_ref[l],
                                 preferred_element_type=jnp.float32)
                         + bm1_ref[l])
        x = x + (jnp.dot(mm, Wm2_ref[l], preferred_element_type=jnp.float32)
                 + bm2_ref[l]).reshape(ROWS, SEG, LATENT)

    xf = _layernorm(x, lnf_g_ref[...], lnf_b_ref[...])
    hlast = xf[B:, SEG - 1, :]
    g = gate_ref[:, pl.ds(i, 1), :].reshape(B, LATENT)
    hg = hlast * g
    mu = jnp.dot(hg, W_mu_ref[...], preferred_element_type=jnp.float32)
    lv = jnp.dot(hg, W_lv_ref[...], preferred_element_type=jnp.float32)
    mu_ref[:, pl.ds(i, 1), :] = (mu + b_mu_ref[...]).reshape(B, 1, LATENT)
    lv_ref[:, pl.ds(i, 1), :] = (lv + b_lv_ref[...]).reshape(B, 1, LATENT)


def _full(shape):
    return pl.BlockSpec(shape, lambda i: (0,) * len(shape))


def kernel(decoder_output, input_ids, segmentation_indices, W_in, b_in,
           ln1_g, ln1_b, W1, b1, conv_k, A_log, W2, b2, ln2_g, ln2_b,
           Wm1, bm1, Wm2, bm2, lnf_g, lnf_b, W_mu, b_mu, W_lv, b_lv):
    gate = segmentation_indices[:, ::SEG, 0].astype(jnp.float32)
    gateb = jnp.broadcast_to(gate[:, :, None], (B, N, LATENT))

    in_specs = [
        pl.BlockSpec((B, SEG, HIDDEN), lambda i: (0, i, 0)),
        pl.BlockSpec((B, SEG, HIDDEN), lambda i: (0, i, 0)),
        _full((B, N, LATENT)),
        _full(W_in.shape), _full(b_in.shape),
        _full(ln1_g.shape), _full(ln1_b.shape),
        _full(W1.shape), _full(b1.shape),
        _full(conv_k.shape), _full(A_log.shape),
        _full(W2.shape), _full(b2.shape),
        _full(ln2_g.shape), _full(ln2_b.shape),
        _full(Wm1.shape), _full(bm1.shape),
        _full(Wm2.shape), _full(bm2.shape),
        _full(lnf_g.shape), _full(lnf_b.shape),
        _full(W_mu.shape), _full(b_mu.shape),
        _full(W_lv.shape), _full(b_lv.shape),
    ]
    out_specs = [_full((B, N, LATENT)), _full((B, N, LATENT))]
    out_shape = [jax.ShapeDtypeStruct((B, N, LATENT), jnp.float32),
                 jax.ShapeDtypeStruct((B, N, LATENT), jnp.float32)]

    mu, lv = pl.pallas_call(
        _mixer_body,
        grid=(N,),
        in_specs=in_specs,
        out_specs=out_specs,
        out_shape=out_shape,
        scratch_shapes=[
            pltpu.VMEM((NUM_LAYERS, B, CONV_LEN - 1, D_INNER), jnp.float32),
            pltpu.VMEM((NUM_LAYERS, B, D_INNER), jnp.float32),
        ],
        compiler_params=pltpu.CompilerParams(
            dimension_semantics=("arbitrary",)),
    )(input_ids, decoder_output, gateb, W_in, b_in, ln1_g, ln1_b, W1, b1,
      conv_k, A_log, W2, b2, ln2_g, ln2_b, Wm1, bm1, Wm2, bm2, lnf_g,
      lnf_b, W_mu, b_mu, W_lv, b_lv)
    return (mu, lv)


# speedup vs baseline: 42.4117x; 1.2753x over previous
import jax
import jax.numpy as jnp
from jax import lax
from jax.experimental import pallas as pl
from jax.experimental.pallas import tpu as pltpu

B, L = 4, 2048
HIDDEN = 512
LATENT = 128
NUM_LAYERS = 2
CONV_LEN = 4
D_INNER = LATENT * 2
MLP_INNER = 256
SEG = 256
N = L // SEG
ROWS = 2 * B
TOK = ROWS * SEG


def _layernorm(x, g, b):
    m = x.mean(-1, keepdims=True)
    xc = x - m
    v = (xc * xc).mean(-1, keepdims=True)
    return xc * lax.rsqrt(v + 1e-5) * g + b


def _silu(x):
    return x * jax.nn.sigmoid(x)


def _mixer_body(ids_ref, dec_ref, gate_ref, W_in_ref, b_in_ref, ln1_g_ref,
                ln1_b_ref, W1_ref, b1_ref, conv_k_ref, A_log_ref, W2_ref,
                b2_ref, ln2_g_ref, ln2_b_ref, Wm1_ref, bm1_ref, Wm2_ref,
                bm2_ref, lnf_g_ref, lnf_b_ref, W_mu_ref, b_mu_ref, W_lv_ref,
                b_lv_ref, mu_ref, lv_ref, tail_scr, ssm_scr):
    i = pl.program_id(0)

    @pl.when(i == 0)
    def _init():
        tail_scr[...] = jnp.zeros_like(tail_scr)
        ssm_scr[...] = jnp.zeros_like(ssm_scr)

    xin = jnp.concatenate([ids_ref[...], dec_ref[...]], axis=0)
    xin = xin.reshape(TOK, HIDDEN)
    x = jnp.dot(xin, W_in_ref[...], preferred_element_type=jnp.float32)
    x = (x + b_in_ref[...]).reshape(ROWS, SEG, LATENT)

    for l in range(NUM_LAYERS):
        last = l == NUM_LAYERS - 1
        h = _layernorm(x, ln1_g_ref[l], ln1_b_ref[l])
        if last:
            u = (jnp.dot(h.reshape(TOK, LATENT), W1_ref[l, :, :D_INNER],
                         preferred_element_type=jnp.float32)
                 + b1_ref[l, :D_INNER]).reshape(ROWS, SEG, D_INNER)
            z_last = (jnp.dot(h[B:, SEG - 1, :], W1_ref[l, :, D_INNER:],
                              preferred_element_type=jnp.float32)
                      + b1_ref[l, D_INNER:])
        else:
            xz = jnp.dot(h.reshape(TOK, LATENT), W1_ref[l],
                         preferred_element_type=jnp.float32) + b1_ref[l]
            xz = xz.reshape(ROWS, SEG, 2 * D_INNER)
            u = xz[:, :, :D_INNER]
            z = xz[:, :, D_INNER:]

        tail = tail_scr[l]
        tail8 = jnp.concatenate([tail, tail], axis=0)
        up = jnp.concatenate([tail8, u], axis=1)
        ck = conv_k_ref[l]
        uc = (up[:, 0:SEG, :] * ck[0] + up[:, 1:SEG + 1, :] * ck[1]
              + up[:, 2:SEG + 2, :] * ck[2] + up[:, 3:SEG + 3, :] * ck[3])
        tail_scr[l] = u[:B, SEG - (CONV_LEN - 1):, :]
        ua = _silu(uc)

        la = jnp.exp(A_log_ref[l])
        sp = ssm_scr[l].reshape(B, 1, D_INNER)
        sp8 = jnp.concatenate([sp, sp], axis=0)
        if last:
            tpow = lax.broadcasted_iota(
                jnp.int32, (SEG, D_INNER), 0).astype(jnp.float32)
            w = jnp.exp((tpow - float(SEG - 1)) * la)
            yT = jnp.sum(ua * w[None, :, :], axis=1)
            yT = yT + jnp.exp(-float(SEG) * la) * sp8.reshape(ROWS, D_INNER)
            ssm_scr[l] = yT[:B]
            y_last = yT[B:]
        else:
            S = ua
            d = 1
            while d < SEG:
                shifted = jnp.concatenate(
                    [jnp.zeros((ROWS, d, D_INNER), jnp.float32),
                     S[:, :SEG - d, :]], axis=1)
                S = S + jnp.exp(-float(d) * la) * shifted
                d *= 2
            tpow = lax.broadcasted_iota(
                jnp.int32, (SEG, D_INNER), 0).astype(jnp.float32) + 1.0
            decay = jnp.exp(-tpow * la)
            y = S + decay[None, :, :] * sp8
            ssm_scr[l] = y[:B, SEG - 1, :]

        if last:
            g2 = y_last * _silu(z_last)
            xd = x[B:, SEG - 1, :] + jnp.dot(
                g2, W2_ref[l], preferred_element_type=jnp.float32) + b2_ref[l]
            h2 = _layernorm(xd, ln2_g_ref[l], ln2_b_ref[l])
            mm = jax.nn.gelu(jnp.dot(h2, Wm1_ref[l],
                                     preferred_element_type=jnp.float32)
                             + bm1_ref[l])
            xd = xd + jnp.dot(mm, Wm2_ref[l],
                              preferred_element_type=jnp.float32) + bm2_ref[l]
        else:
            g2 = (y * _silu(z)).reshape(TOK, D_INNER)
            x = x + (jnp.dot(g2, W2_ref[l],
                             preferred_element_type=jnp.float32)
                     + b2_ref[l]).reshape(ROWS, SEG, LATENT)
            h2 = _layernorm(x, ln2_g_ref[l], ln2_b_ref[l])
            mm = jax.nn.gelu(jnp.dot(h2.reshape(TOK, LATENT), Wm1_ref[l],
                                     preferred_element_type=jnp.float32)
                             + bm1_ref[l])
            x = x + (jnp.dot(mm, Wm2_ref[l],
                             preferred_element_type=jnp.float32)
                     + bm2_ref[l]).reshape(ROWS, SEG, LATENT)

    hlast = _layernorm(xd, lnf_g_ref[...], lnf_b_ref[...])
    g = gate_ref[:, pl.ds(i, 1), :].reshape(B, LATENT)
    hg = hlast * g
    mu = jnp.dot(hg, W_mu_ref[...], preferred_element_type=jnp.float32)
    lv = jnp.dot(hg, W_lv_ref[...], preferred_element_type=jnp.float32)
    mu_ref[:, pl.ds(i, 1), :] = (mu + b_mu_ref[...]).reshape(B, 1, LATENT)
    lv_ref[:, pl.ds(i, 1), :] = (lv + b_lv_ref[...]).reshape(B, 1, LATENT)


def _full(shape):
    return pl.BlockSpec(shape, lambda i: (0,) * len(shape))


def kernel(decoder_output, input_ids, segmentation_indices, W_in, b_in,
           ln1_g, ln1_b, W1, b1, conv_k, A_log, W2, b2, ln2_g, ln2_b,
           Wm1, bm1, Wm2, bm2, lnf_g, lnf_b, W_mu, b_mu, W_lv, b_lv):
    gate = segmentation_indices[:, ::SEG, 0].astype(jnp.float32)
    gateb = jnp.broadcast_to(gate[:, :, None], (B, N, LATENT))

    in_specs = [
        pl.BlockSpec((B, SEG, HIDDEN), lambda i: (0, i, 0)),
        pl.BlockSpec((B, SEG, HIDDEN), lambda i: (0, i, 0)),
        _full((B, N, LATENT)),
        _full(W_in.shape), _full(b_in.shape),
        _full(ln1_g.shape), _full(ln1_b.shape),
        _full(W1.shape), _full(b1.shape),
        _full(conv_k.shape), _full(A_log.shape),
        _full(W2.shape), _full(b2.shape),
        _full(ln2_g.shape), _full(ln2_b.shape),
        _full(Wm1.shape), _full(bm1.shape),
        _full(Wm2.shape), _full(bm2.shape),
        _full(lnf_g.shape), _full(lnf_b.shape),
        _full(W_mu.shape), _full(b_mu.shape),
        _full(W_lv.shape), _full(b_lv.shape),
    ]
    out_specs = [_full((B, N, LATENT)), _full((B, N, LATENT))]
    out_shape = [jax.ShapeDtypeStruct((B, N, LATENT), jnp.float32),
                 jax.ShapeDtypeStruct((B, N, LATENT), jnp.float32)]

    mu, lv = pl.pallas_call(
        _mixer_body,
        grid=(N,),
        in_specs=in_specs,
        out_specs=out_specs,
        out_shape=out_shape,
        scratch_shapes=[
            pltpu.VMEM((NUM_LAYERS, B, CONV_LEN - 1, D_INNER), jnp.float32),
            pltpu.VMEM((NUM_LAYERS, B, D_INNER), jnp.float32),
        ],
        compiler_params=pltpu.CompilerParams(
            dimension_semantics=("arbitrary",)),
    )(input_ids, decoder_output, gateb, W_in, b_in, ln1_g, ln1_b, W1, b1,
      conv_k, A_log, W2, b2, ln2_g, ln2_b, Wm1, bm1, Wm2, bm2, lnf_g,
      lnf_b, W_mu, b_mu, W_lv, b_lv)
    return (mu, lv)


# speedup vs baseline: 50.3427x; 1.1870x over previous
import jax
import jax.numpy as jnp
from jax import lax
from jax.experimental import pallas as pl
from jax.experimental.pallas import tpu as pltpu

B, L = 4, 2048
HIDDEN = 512
LATENT = 128
NUM_LAYERS = 2
CONV_LEN = 4
D_INNER = LATENT * 2
MLP_INNER = 256
SEG = 256
N = L // SEG
ROWS = 2 * B
TOK = ROWS * SEG


def _layernorm(x, g, b):
    m = x.mean(-1, keepdims=True)
    xc = x - m
    v = (xc * xc).mean(-1, keepdims=True)
    return xc * lax.rsqrt(v + 1e-5) * g + b


def _silu(x):
    return x * jax.nn.sigmoid(x)


def _mixer_body(ids_ref, dec_ref, gate_ref, W_in_ref, b_in_ref, ln1_g_ref,
                ln1_b_ref, W1_ref, b1_ref, conv_k_ref, A_log_ref, W2_ref,
                b2_ref, ln2_g_ref, ln2_b_ref, Wm1_ref, bm1_ref, Wm2_ref,
                bm2_ref, lnf_g_ref, lnf_b_ref, W_mu_ref, b_mu_ref, W_lv_ref,
                b_lv_ref, mu_ref, lv_ref, tail_scr, ssm_scr):
    i = pl.program_id(0)

    @pl.when(i == 0)
    def _init():
        tail_scr[...] = jnp.zeros_like(tail_scr)
        ssm_scr[...] = jnp.zeros_like(ssm_scr)

    xin = jnp.concatenate([ids_ref[...], dec_ref[...]], axis=0)
    xin = xin.reshape(TOK, HIDDEN)
    x = jnp.dot(xin, W_in_ref[...], preferred_element_type=jnp.float32)
    x = (x + b_in_ref[...]).reshape(ROWS, SEG, LATENT)

    for l in range(NUM_LAYERS):
        last = l == NUM_LAYERS - 1
        h = _layernorm(x, ln1_g_ref[l], ln1_b_ref[l])
        h2d = h.reshape(TOK, LATENT)
        W1u = W1_ref[l, :, :D_INNER]
        b1u = b1_ref[l, :D_INNER]
        ck = conv_k_ref[l]

        shifts = [h]
        for j in range(1, CONV_LEN):
            shifts.append(jnp.concatenate(
                [jnp.zeros((ROWS, j, LATENT), jnp.float32),
                 h[:, :SEG - j, :]], axis=1))
        h4 = jnp.concatenate(shifts, axis=2).reshape(TOK, CONV_LEN * LATENT)
        sw4 = jnp.concatenate(
            [W1u * ck[CONV_LEN - 1 - j] for j in range(CONV_LEN)], axis=0)
        core = jnp.dot(h4, sw4, preferred_element_type=jnp.float32)
        cb = (ck[0] + ck[1] + ck[2] + ck[3]) * b1u
        uc = core.reshape(ROWS, SEG, D_INNER) + cb
        tail = tail_scr[l]
        tq0 = tail[:, 5, :] - b1u
        tq1 = tail[:, 6, :] - b1u
        tq2 = tail[:, 7, :] - b1u
        f0 = ck[0] * tq0 + ck[1] * tq1 + ck[2] * tq2
        f1 = ck[0] * tq1 + ck[1] * tq2
        f2 = ck[0] * tq2
        fix = jnp.concatenate(
            [f0.reshape(B, 1, D_INNER), f1.reshape(B, 1, D_INNER),
             f2.reshape(B, 1, D_INNER),
             jnp.zeros((B, 5, D_INNER), jnp.float32)], axis=1)
        fix8 = jnp.concatenate([fix, fix], axis=0)
        uc = jnp.concatenate([uc[:, :8, :] + fix8, uc[:, 8:, :]], axis=1)
        hl8 = h[:B, SEG - 8:, :].reshape(8 * B, LATENT)
        tail_scr[l] = (jnp.dot(hl8, W1u, preferred_element_type=jnp.float32)
                       + b1u).reshape(B, 8, D_INNER)
        ua = _silu(uc)

        la = jnp.exp(A_log_ref[l])
        sp = ssm_scr[l].reshape(B, 1, D_INNER)
        sp8 = jnp.concatenate([sp, sp], axis=0)
        if last:
            tpow = lax.broadcasted_iota(
                jnp.int32, (SEG, D_INNER), 0).astype(jnp.float32)
            w = jnp.exp((tpow - float(SEG - 1)) * la)
            yT = jnp.sum(ua * w[None, :, :], axis=1)
            yT = yT + jnp.exp(-float(SEG) * la) * sp8.reshape(ROWS, D_INNER)
            ssm_scr[l] = yT[:B]
            y_last = yT[B:]
        else:
            TB = 16
            NB = SEG // TB
            S = ua.reshape(ROWS, NB, TB, D_INNER)
            d = 1
            while d < TB:
                shifted = jnp.concatenate(
                    [jnp.zeros((ROWS, NB, d, D_INNER), jnp.float32),
                     S[:, :, :TB - d, :]], axis=2)
                S = S + jnp.exp(-float(d) * la) * shifted
                d *= 2
            c = S[:, :, TB - 1, :]
            d = 1
            while d < NB:
                shifted = jnp.concatenate(
                    [jnp.zeros((ROWS, d, D_INNER), jnp.float32),
                     c[:, :NB - d, :]], axis=1)
                c = c + jnp.exp(-float(TB * d) * la) * shifted
                d *= 2
            mpow = lax.broadcasted_iota(
                jnp.int32, (NB, D_INNER), 0).astype(jnp.float32) + 1.0
            c = c + jnp.exp(-float(TB) * mpow * la)[None] * sp8
            ssm_scr[l] = c[:B, NB - 1, :]
            carry = jnp.concatenate([sp8, c[:, :NB - 1, :]], axis=1)
            tpow = lax.broadcasted_iota(
                jnp.int32, (TB, D_INNER), 0).astype(jnp.float32) + 1.0
            dec16 = jnp.exp(-tpow * la)
            y = S + dec16[None, None] * carry[:, :, None, :]
            y = y.reshape(ROWS, SEG, D_INNER)

        if last:
            z_last = (jnp.dot(h[B:, SEG - 1, :], W1_ref[l, :, D_INNER:],
                              preferred_element_type=jnp.float32)
                      + b1_ref[l, D_INNER:])
            g2 = y_last * _silu(z_last)
            xd = x[B:, SEG - 1, :] + jnp.dot(
                g2, W2_ref[l], preferred_element_type=jnp.float32) + b2_ref[l]
            h2 = _layernorm(xd, ln2_g_ref[l], ln2_b_ref[l])
            mm = jax.nn.gelu(jnp.dot(h2, Wm1_ref[l],
                                     preferred_element_type=jnp.float32)
                             + bm1_ref[l])
            xd = xd + jnp.dot(mm, Wm2_ref[l],
                              preferred_element_type=jnp.float32) + bm2_ref[l]
        else:
            z = (jnp.dot(h2d, W1_ref[l, :, D_INNER:],
                         preferred_element_type=jnp.float32)
                 + b1_ref[l, D_INNER:]).reshape(ROWS, SEG, D_INNER)
            g2 = (y * _silu(z)).reshape(TOK, D_INNER)
            x = x + (jnp.dot(g2, W2_ref[l],
                             preferred_element_type=jnp.float32)
                     + b2_ref[l]).reshape(ROWS, SEG, LATENT)
            h2 = _layernorm(x, ln2_g_ref[l], ln2_b_ref[l])
            mm = jax.nn.gelu(jnp.dot(h2.reshape(TOK, LATENT), Wm1_ref[l],
                                     preferred_element_type=jnp.float32)
                             + bm1_ref[l])
            x = x + (jnp.dot(mm, Wm2_ref[l],
                             preferred_element_type=jnp.float32)
                     + bm2_ref[l]).reshape(ROWS, SEG, LATENT)

    hlast = _layernorm(xd, lnf_g_ref[...], lnf_b_ref[...])
    g = gate_ref[:, pl.ds(i, 1), :].reshape(B, LATENT)
    hg = hlast * g
    mu = jnp.dot(hg, W_mu_ref[...], preferred_element_type=jnp.float32)
    lv = jnp.dot(hg, W_lv_ref[...], preferred_element_type=jnp.float32)
    mu_ref[:, pl.ds(i, 1), :] = (mu + b_mu_ref[...]).reshape(B, 1, LATENT)
    lv_ref[:, pl.ds(i, 1), :] = (lv + b_lv_ref[...]).reshape(B, 1, LATENT)


def _full(shape):
    return pl.BlockSpec(shape, lambda i: (0,) * len(shape))


def kernel(decoder_output, input_ids, segmentation_indices, W_in, b_in,
           ln1_g, ln1_b, W1, b1, conv_k, A_log, W2, b2, ln2_g, ln2_b,
           Wm1, bm1, Wm2, bm2, lnf_g, lnf_b, W_mu, b_mu, W_lv, b_lv):
    gate = segmentation_indices[:, ::SEG, 0].astype(jnp.float32)
    gateb = jnp.broadcast_to(gate[:, :, None], (B, N, LATENT))

    in_specs = [
        pl.BlockSpec((B, SEG, HIDDEN), lambda i: (0, i, 0)),
        pl.BlockSpec((B, SEG, HIDDEN), lambda i: (0, i, 0)),
        _full((B, N, LATENT)),
        _full(W_in.shape), _full(b_in.shape),
        _full(ln1_g.shape), _full(ln1_b.shape),
        _full(W1.shape), _full(b1.shape),
        _full(conv_k.shape), _full(A_log.shape),
        _full(W2.shape), _full(b2.shape),
        _full(ln2_g.shape), _full(ln2_b.shape),
        _full(Wm1.shape), _full(bm1.shape),
        _full(Wm2.shape), _full(bm2.shape),
        _full(lnf_g.shape), _full(lnf_b.shape),
        _full(W_mu.shape), _full(b_mu.shape),
        _full(W_lv.shape), _full(b_lv.shape),
    ]
    out_specs = [_full((B, N, LATENT)), _full((B, N, LATENT))]
    out_shape = [jax.ShapeDtypeStruct((B, N, LATENT), jnp.float32),
                 jax.ShapeDtypeStruct((B, N, LATENT), jnp.float32)]

    mu, lv = pl.pallas_call(
        _mixer_body,
        grid=(N,),
        in_specs=in_specs,
        out_specs=out_specs,
        out_shape=out_shape,
        scratch_shapes=[
            pltpu.VMEM((NUM_LAYERS, B, 8, D_INNER), jnp.float32),
            pltpu.VMEM((NUM_LAYERS, B, D_INNER), jnp.float32),
        ],
        compiler_params=pltpu.CompilerParams(
            dimension_semantics=("arbitrary",)),
    )(input_ids, decoder_output, gateb, W_in, b_in, ln1_g, ln1_b, W1, b1,
      conv_k, A_log, W2, b2, ln2_g, ln2_b, Wm1, bm1, Wm2, bm2, lnf_g,
      lnf_b, W_mu, b_mu, W_lv, b_lv)
    return (mu, lv)


# speedup vs baseline: 52.3763x; 1.0404x over previous
import jax
import jax.numpy as jnp
from jax import lax
from jax.experimental import pallas as pl
from jax.experimental.pallas import tpu as pltpu

B, L = 4, 2048
HIDDEN = 512
LATENT = 128
NUM_LAYERS = 2
CONV_LEN = 4
D_INNER = LATENT * 2
MLP_INNER = 256
SEG = 256
N = L // SEG
ROWS = 2 * B
TOK = ROWS * SEG


def _layernorm(x, g, b):
    m = x.mean(-1, keepdims=True)
    xc = x - m
    v = (xc * xc).mean(-1, keepdims=True)
    return xc * lax.rsqrt(v + 1e-5) * g + b


def _silu(x):
    return x * jax.nn.sigmoid(x)


def _mixer_body(ids_ref, dec_ref, gate_ref, W_in_ref, b_in_ref, ln1_g_ref,
                ln1_b_ref, W1_ref, b1_ref, conv_k_ref, A_log_ref, W2_ref,
                b2_ref, ln2_g_ref, ln2_b_ref, Wm1_ref, bm1_ref, Wm2_ref,
                bm2_ref, lnf_g_ref, lnf_b_ref, W_mu_ref, b_mu_ref, W_lv_ref,
                b_lv_ref, mu_ref, lv_ref, tail_scr, ssm_scr):
    i = pl.program_id(0)

    @pl.when(i == 0)
    def _init():
        tail_scr[...] = jnp.zeros_like(tail_scr)
        ssm_scr[...] = jnp.zeros_like(ssm_scr)

    Win = W_in_ref[...]
    xc = jnp.dot(ids_ref[...].reshape(B * SEG, HIDDEN), Win,
                 preferred_element_type=jnp.float32)
    xd = jnp.dot(dec_ref[...].reshape(B * SEG, HIDDEN), Win,
                 preferred_element_type=jnp.float32)
    x = (jnp.concatenate([xc, xd], axis=0)
         + b_in_ref[...]).reshape(ROWS, SEG, LATENT)

    for l in range(NUM_LAYERS):
        last = l == NUM_LAYERS - 1
        h = _layernorm(x, ln1_g_ref[l], ln1_b_ref[l])
        h2d = h.reshape(TOK, LATENT)
        W1u = W1_ref[l, :, :D_INNER]
        b1u = b1_ref[l, :D_INNER]
        ck = conv_k_ref[l]

        shifts = [h]
        for j in range(1, CONV_LEN):
            shifts.append(jnp.concatenate(
                [jnp.zeros((ROWS, j, LATENT), jnp.float32),
                 h[:, :SEG - j, :]], axis=1))
        h4 = jnp.concatenate(shifts, axis=2).reshape(TOK, CONV_LEN * LATENT)
        sw4 = jnp.concatenate(
            [W1u * ck[CONV_LEN - 1 - j] for j in range(CONV_LEN)], axis=0)
        core = jnp.dot(h4, sw4, preferred_element_type=jnp.float32)
        cb = (ck[0] + ck[1] + ck[2] + ck[3]) * b1u
        uc = core.reshape(ROWS, SEG, D_INNER) + cb
        tail = tail_scr[l]
        tq0 = tail[:, 5, :] - b1u
        tq1 = tail[:, 6, :] - b1u
        tq2 = tail[:, 7, :] - b1u
        f0 = ck[0] * tq0 + ck[1] * tq1 + ck[2] * tq2
        f1 = ck[0] * tq1 + ck[1] * tq2
        f2 = ck[0] * tq2
        fix = jnp.concatenate(
            [f0.reshape(B, 1, D_INNER), f1.reshape(B, 1, D_INNER),
             f2.reshape(B, 1, D_INNER),
             jnp.zeros((B, 5, D_INNER), jnp.float32)], axis=1)
        fix8 = jnp.concatenate([fix, fix], axis=0)
        uc = jnp.concatenate([uc[:, :8, :] + fix8, uc[:, 8:, :]], axis=1)
        hl8 = h[:B, SEG - 8:, :].reshape(8 * B, LATENT)
        tail_scr[l] = (jnp.dot(hl8, W1u, preferred_element_type=jnp.float32)
                       + b1u).reshape(B, 8, D_INNER)
        ua = _silu(uc)

        la = jnp.exp(A_log_ref[l])
        sp = ssm_scr[l].reshape(B, 1, D_INNER)
        sp8 = jnp.concatenate([sp, sp], axis=0)
        if last:
            tpow = lax.broadcasted_iota(
                jnp.int32, (SEG, D_INNER), 0).astype(jnp.float32)
            w = jnp.exp((tpow - float(SEG - 1)) * la)
            yT = jnp.sum(ua * w[None, :, :], axis=1)
            yT = yT + jnp.exp(-float(SEG) * la) * sp8.reshape(ROWS, D_INNER)
            ssm_scr[l] = yT[:B]
            y_last = yT[B:]
        else:
            TB = 16
            NB = SEG // TB
            S = ua.reshape(ROWS, NB, TB, D_INNER)
            d = 1
            while d < TB:
                shifted = jnp.concatenate(
                    [jnp.zeros((ROWS, NB, d, D_INNER), jnp.float32),
                     S[:, :, :TB - d, :]], axis=2)
                S = S + jnp.exp(-float(d) * la) * shifted
                d *= 2
            c = S[:, :, TB - 1, :]
            d = 1
            while d < NB:
                shifted = jnp.concatenate(
                    [jnp.zeros((ROWS, d, D_INNER), jnp.float32),
                     c[:, :NB - d, :]], axis=1)
                c = c + jnp.exp(-float(TB * d) * la) * shifted
                d *= 2
            mpow = lax.broadcasted_iota(
                jnp.int32, (NB, D_INNER), 0).astype(jnp.float32) + 1.0
            c = c + jnp.exp(-float(TB) * mpow * la)[None] * sp8
            ssm_scr[l] = c[:B, NB - 1, :]
            carry = jnp.concatenate([sp8, c[:, :NB - 1, :]], axis=1)
            tpow = lax.broadcasted_iota(
                jnp.int32, (TB, D_INNER), 0).astype(jnp.float32) + 1.0
            dec16 = jnp.exp(-tpow * la)
            y = S + dec16[None, None] * carry[:, :, None, :]
            y = y.reshape(ROWS, SEG, D_INNER)

        if last:
            z_last = (jnp.dot(h[B:, SEG - 1, :], W1_ref[l, :, D_INNER:],
                              preferred_element_type=jnp.float32)
                      + b1_ref[l, D_INNER:])
            g2 = y_last * _silu(z_last)
            xd = x[B:, SEG - 1, :] + jnp.dot(
                g2, W2_ref[l], preferred_element_type=jnp.float32) + b2_ref[l]
            h2 = _layernorm(xd, ln2_g_ref[l], ln2_b_ref[l])
            mm = jax.nn.gelu(jnp.dot(h2, Wm1_ref[l],
                                     preferred_element_type=jnp.float32)
                             + bm1_ref[l])
            xd = xd + jnp.dot(mm, Wm2_ref[l],
                              preferred_element_type=jnp.float32) + bm2_ref[l]
        else:
            z = (jnp.dot(h2d, W1_ref[l, :, D_INNER:],
                         preferred_element_type=jnp.float32)
                 + b1_ref[l, D_INNER:]).reshape(ROWS, SEG, D_INNER)
            g2 = (y * _silu(z)).reshape(TOK, D_INNER)
            x = x + (jnp.dot(g2, W2_ref[l],
                             preferred_element_type=jnp.float32)
                     + b2_ref[l]).reshape(ROWS, SEG, LATENT)
            h2 = _layernorm(x, ln2_g_ref[l], ln2_b_ref[l])
            mm = jax.nn.gelu(jnp.dot(h2.reshape(TOK, LATENT), Wm1_ref[l],
                                     preferred_element_type=jnp.float32)
                             + bm1_ref[l])
            x = x + (jnp.dot(mm, Wm2_ref[l],
                             preferred_element_type=jnp.float32)
                     + bm2_ref[l]).reshape(ROWS, SEG, LATENT)

    hlast = _layernorm(xd, lnf_g_ref[...], lnf_b_ref[...])
    g = gate_ref[:, pl.ds(i, 1), :].reshape(B, LATENT)
    hg = hlast * g
    mu = jnp.dot(hg, W_mu_ref[...], preferred_element_type=jnp.float32)
    lv = jnp.dot(hg, W_lv_ref[...], preferred_element_type=jnp.float32)
    mu_ref[:, pl.ds(i, 1), :] = (mu + b_mu_ref[...]).reshape(B, 1, LATENT)
    lv_ref[:, pl.ds(i, 1), :] = (lv + b_lv_ref[...]).reshape(B, 1, LATENT)


def _full(shape):
    return pl.BlockSpec(shape, lambda i: (0,) * len(shape))


def kernel(decoder_output, input_ids, segmentation_indices, W_in, b_in,
           ln1_g, ln1_b, W1, b1, conv_k, A_log, W2, b2, ln2_g, ln2_b,
           Wm1, bm1, Wm2, bm2, lnf_g, lnf_b, W_mu, b_mu, W_lv, b_lv):
    gate = segmentation_indices[:, ::SEG, 0].astype(jnp.float32)
    gateb = jnp.broadcast_to(gate[:, :, None], (B, N, LATENT))

    in_specs = [
        pl.BlockSpec((B, SEG, HIDDEN), lambda i: (0, i, 0)),
        pl.BlockSpec((B, SEG, HIDDEN), lambda i: (0, i, 0)),
        _full((B, N, LATENT)),
        _full(W_in.shape), _full(b_in.shape),
        _full(ln1_g.shape), _full(ln1_b.shape),
        _full(W1.shape), _full(b1.shape),
        _full(conv_k.shape), _full(A_log.shape),
        _full(W2.shape), _full(b2.shape),
        _full(ln2_g.shape), _full(ln2_b.shape),
        _full(Wm1.shape), _full(bm1.shape),
        _full(Wm2.shape), _full(bm2.shape),
        _full(lnf_g.shape), _full(lnf_b.shape),
        _full(W_mu.shape), _full(b_mu.shape),
        _full(W_lv.shape), _full(b_lv.shape),
    ]
    out_specs = [_full((B, N, LATENT)), _full((B, N, LATENT))]
    out_shape = [jax.ShapeDtypeStruct((B, N, LATENT), jnp.float32),
                 jax.ShapeDtypeStruct((B, N, LATENT), jnp.float32)]

    mu, lv = pl.pallas_call(
        _mixer_body,
        grid=(N,),
        in_specs=in_specs,
        out_specs=out_specs,
        out_shape=out_shape,
        scratch_shapes=[
            pltpu.VMEM((NUM_LAYERS, B, 8, D_INNER), jnp.float32),
            pltpu.VMEM((NUM_LAYERS, B, D_INNER), jnp.float32),
        ],
        compiler_params=pltpu.CompilerParams(
            dimension_semantics=("arbitrary",)),
    )(input_ids, decoder_output, gateb, W_in, b_in, ln1_g, ln1_b, W1, b1,
      conv_k, A_log, W2, b2, ln2_g, ln2_b, Wm1, bm1, Wm2, bm2, lnf_g,
      lnf_b, W_mu, b_mu, W_lv, b_lv)
    return (mu, lv)


# speedup vs baseline: 57.8241x; 1.1040x over previous
import jax
import jax.numpy as jnp
from jax import lax
from jax.experimental import pallas as pl
from jax.experimental.pallas import tpu as pltpu

B, L = 4, 2048
HIDDEN = 512
LATENT = 128
NUM_LAYERS = 2
CONV_LEN = 4
D_INNER = LATENT * 2
MLP_INNER = 256
SEG = 256
N = L // SEG
ROWS = 2 * B
TOK = ROWS * SEG


def _layernorm(x, g, b):
    m = x.mean(-1, keepdims=True)
    xc = x - m
    v = (xc * xc).mean(-1, keepdims=True)
    return xc * lax.rsqrt(v + 1e-5) * g + b


def _silu(x):
    return x * jax.nn.sigmoid(x)


def _mixer_body(ids_ref, dec_ref, gate_ref, W_in_ref, b_in_ref, ln1_g_ref,
                ln1_b_ref, W1_ref, b1_ref, conv_k_ref, A_log_ref, W2_ref,
                b2_ref, ln2_g_ref, ln2_b_ref, Wm1_ref, bm1_ref, Wm2_ref,
                bm2_ref, lnf_g_ref, lnf_b_ref, W_mu_ref, b_mu_ref, W_lv_ref,
                b_lv_ref, mu_ref, lv_ref, tail_scr, ssm_scr, xl_scr,
                hl_scr, yl_scr):
    i = pl.program_id(0)

    @pl.when(i == 0)
    def _init():
        tail_scr[...] = jnp.zeros_like(tail_scr)
        ssm_scr[...] = jnp.zeros_like(ssm_scr)

    Win = W_in_ref[...]
    xc = jnp.dot(ids_ref[...].reshape(B * SEG, HIDDEN), Win,
                 preferred_element_type=jnp.float32)
    xd = jnp.dot(dec_ref[...].reshape(B * SEG, HIDDEN), Win,
                 preferred_element_type=jnp.float32)
    x = (jnp.concatenate([xc, xd], axis=0)
         + b_in_ref[...]).reshape(ROWS, SEG, LATENT)

    for l in range(NUM_LAYERS):
        last = l == NUM_LAYERS - 1
        h = _layernorm(x, ln1_g_ref[l], ln1_b_ref[l])
        h2d = h.reshape(TOK, LATENT)
        W1u = W1_ref[l, :, :D_INNER]
        b1u = b1_ref[l, :D_INNER]
        ck = conv_k_ref[l]

        shifts = [h]
        for j in range(1, CONV_LEN):
            shifts.append(jnp.concatenate(
                [jnp.zeros((ROWS, j, LATENT), jnp.float32),
                 h[:, :SEG - j, :]], axis=1))
        h4 = jnp.concatenate(shifts, axis=2).reshape(TOK, CONV_LEN * LATENT)
        sw4 = jnp.concatenate(
            [W1u * ck[CONV_LEN - 1 - j] for j in range(CONV_LEN)], axis=0)
        core = jnp.dot(h4, sw4, preferred_element_type=jnp.float32)
        cb = (ck[0] + ck[1] + ck[2] + ck[3]) * b1u
        uc = core.reshape(ROWS, SEG, D_INNER) + cb
        tail = tail_scr[l]
        tq0 = tail[:, 5, :] - b1u
        tq1 = tail[:, 6, :] - b1u
        tq2 = tail[:, 7, :] - b1u
        f0 = ck[0] * tq0 + ck[1] * tq1 + ck[2] * tq2
        f1 = ck[0] * tq1 + ck[1] * tq2
        f2 = ck[0] * tq2
        fix = jnp.concatenate(
            [f0.reshape(B, 1, D_INNER), f1.reshape(B, 1, D_INNER),
             f2.reshape(B, 1, D_INNER),
             jnp.zeros((B, 5, D_INNER), jnp.float32)], axis=1)
        fix8 = jnp.concatenate([fix, fix], axis=0)
        uc = jnp.concatenate([uc[:, :8, :] + fix8, uc[:, 8:, :]], axis=1)
        hl8 = h[:B, SEG - 8:, :].reshape(8 * B, LATENT)
        tail_scr[l] = (jnp.dot(hl8, W1u, preferred_element_type=jnp.float32)
                       + b1u).reshape(B, 8, D_INNER)
        ua = _silu(uc)

        la = jnp.exp(A_log_ref[l])
        sp = ssm_scr[l].reshape(B, 1, D_INNER)
        sp8 = jnp.concatenate([sp, sp], axis=0)
        if last:
            tpow = lax.broadcasted_iota(
                jnp.int32, (SEG, D_INNER), 0).astype(jnp.float32)
            w = jnp.exp((tpow - float(SEG - 1)) * la)
            yT = jnp.sum(ua * w[None, :, :], axis=1)
            yT = yT + jnp.exp(-float(SEG) * la) * sp8.reshape(ROWS, D_INNER)
            ssm_scr[l] = yT[:B]
            y_last = yT[B:]
        else:
            TB = 16
            NB = SEG // TB
            S = ua.reshape(ROWS, NB, TB, D_INNER)
            d = 1
            while d < TB:
                shifted = jnp.concatenate(
                    [jnp.zeros((ROWS, NB, d, D_INNER), jnp.float32),
                     S[:, :, :TB - d, :]], axis=2)
                S = S + jnp.exp(-float(d) * la) * shifted
                d *= 2
            c = S[:, :, TB - 1, :]
            d = 1
            while d < NB:
                shifted = jnp.concatenate(
                    [jnp.zeros((ROWS, d, D_INNER), jnp.float32),
                     c[:, :NB - d, :]], axis=1)
                c = c + jnp.exp(-float(TB * d) * la) * shifted
                d *= 2
            mpow = lax.broadcasted_iota(
                jnp.int32, (NB, D_INNER), 0).astype(jnp.float32) + 1.0
            c = c + jnp.exp(-float(TB) * mpow * la)[None] * sp8
            ssm_scr[l] = c[:B, NB - 1, :]
            carry = jnp.concatenate([sp8, c[:, :NB - 1, :]], axis=1)
            tpow = lax.broadcasted_iota(
                jnp.int32, (TB, D_INNER), 0).astype(jnp.float32) + 1.0
            dec16 = jnp.exp(-tpow * la)
            y = S + dec16[None, None] * carry[:, :, None, :]
            y = y.reshape(ROWS, SEG, D_INNER)

        if last:
            xl_scr[pl.ds(i, 1)] = x[B:, SEG - 1, :].reshape(1, B, LATENT)
            hl_scr[pl.ds(i, 1)] = h[B:, SEG - 1, :].reshape(1, B, LATENT)
            yl_scr[pl.ds(i, 1)] = y_last.reshape(1, B, D_INNER)
        else:
            z = (jnp.dot(h2d, W1_ref[l, :, D_INNER:],
                         preferred_element_type=jnp.float32)
                 + b1_ref[l, D_INNER:]).reshape(ROWS, SEG, D_INNER)
            g2 = (y * _silu(z)).reshape(TOK, D_INNER)
            x = x + (jnp.dot(g2, W2_ref[l],
                             preferred_element_type=jnp.float32)
                     + b2_ref[l]).reshape(ROWS, SEG, LATENT)
            h2 = _layernorm(x, ln2_g_ref[l], ln2_b_ref[l])
            mm = jax.nn.gelu(jnp.dot(h2.reshape(TOK, LATENT), Wm1_ref[l],
                                     preferred_element_type=jnp.float32)
                             + bm1_ref[l])
            x = x + (jnp.dot(mm, Wm2_ref[l],
                             preferred_element_type=jnp.float32)
                     + bm2_ref[l]).reshape(ROWS, SEG, LATENT)

    @pl.when(i == N - 1)
    def _tail():
        lo = NUM_LAYERS - 1
        xl = xl_scr[...].reshape(N * B, LATENT)
        hl = hl_scr[...].reshape(N * B, LATENT)
        yl = yl_scr[...].reshape(N * B, D_INNER)
        z = (jnp.dot(hl, W1_ref[lo, :, D_INNER:],
                     preferred_element_type=jnp.float32)
             + b1_ref[lo, D_INNER:])
        g2 = yl * _silu(z)
        xd = xl + jnp.dot(g2, W2_ref[lo],
                          preferred_element_type=jnp.float32) + b2_ref[lo]
        h2 = _layernorm(xd, ln2_g_ref[lo], ln2_b_ref[lo])
        mm = jax.nn.gelu(jnp.dot(h2, Wm1_ref[lo],
                                 preferred_element_type=jnp.float32)
                         + bm1_ref[lo])
        xd = xd + jnp.dot(mm, Wm2_ref[lo],
                          preferred_element_type=jnp.float32) + bm2_ref[lo]
        hg = _layernorm(xd, lnf_g_ref[...], lnf_b_ref[...])
        hg = hg * gate_ref[...].reshape(N * B, LATENT)
        mu = (jnp.dot(hg, W_mu_ref[...], preferred_element_type=jnp.float32)
              + b_mu_ref[...]).reshape(N, B, LATENT)
        lv = (jnp.dot(hg, W_lv_ref[...], preferred_element_type=jnp.float32)
              + b_lv_ref[...]).reshape(N, B, LATENT)
        for b in range(B):
            mu_ref[b] = mu[:, b, :]
            lv_ref[b] = lv[:, b, :]


def _full(shape):
    return pl.BlockSpec(shape, lambda i: (0,) * len(shape))


def kernel(decoder_output, input_ids, segmentation_indices, W_in, b_in,
           ln1_g, ln1_b, W1, b1, conv_k, A_log, W2, b2, ln2_g, ln2_b,
           Wm1, bm1, Wm2, bm2, lnf_g, lnf_b, W_mu, b_mu, W_lv, b_lv):
    gate = segmentation_indices[:, ::SEG, 0].astype(jnp.float32)
    gateb = jnp.broadcast_to(gate.T[:, :, None], (N, B, LATENT))

    in_specs = [
        pl.BlockSpec((B, SEG, HIDDEN), lambda i: (0, i, 0)),
        pl.BlockSpec((B, SEG, HIDDEN), lambda i: (0, i, 0)),
        _full((N, B, LATENT)),
        _full(W_in.shape), _full(b_in.shape),
        _full(ln1_g.shape), _full(ln1_b.shape),
        _full(W1.shape), _full(b1.shape),
        _full(conv_k.shape), _full(A_log.shape),
        _full(W2.shape), _full(b2.shape),
        _full(ln2_g.shape), _full(ln2_b.shape),
        _full(Wm1.shape), _full(bm1.shape),
        _full(Wm2.shape), _full(bm2.shape),
        _full(lnf_g.shape), _full(lnf_b.shape),
        _full(W_mu.shape), _full(b_mu.shape),
        _full(W_lv.shape), _full(b_lv.shape),
    ]
    out_specs = [_full((B, N, LATENT)), _full((B, N, LATENT))]
    out_shape = [jax.ShapeDtypeStruct((B, N, LATENT), jnp.float32),
                 jax.ShapeDtypeStruct((B, N, LATENT), jnp.float32)]

    mu, lv = pl.pallas_call(
        _mixer_body,
        grid=(N,),
        in_specs=in_specs,
        out_specs=out_specs,
        out_shape=out_shape,
        scratch_shapes=[
            pltpu.VMEM((NUM_LAYERS, B, 8, D_INNER), jnp.float32),
            pltpu.VMEM((NUM_LAYERS, B, D_INNER), jnp.float32),
            pltpu.VMEM((N, B, LATENT), jnp.float32),
            pltpu.VMEM((N, B, LATENT), jnp.float32),
            pltpu.VMEM((N, B, D_INNER), jnp.float32),
        ],
        compiler_params=pltpu.CompilerParams(
            dimension_semantics=("arbitrary",)),
    )(input_ids, decoder_output, gateb, W_in, b_in, ln1_g, ln1_b, W1, b1,
      conv_k, A_log, W2, b2, ln2_g, ln2_b, Wm1, bm1, Wm2, bm2, lnf_g,
      lnf_b, W_mu, b_mu, W_lv, b_lv)
    return (mu, lv)
